# baseline (speedup 1.0000x reference)
import functools

import jax
import jax.numpy as jnp
from jax import lax
from jax.experimental import pallas as pl
from jax.experimental.pallas import tpu as pltpu
from jax.experimental.pallas import tpu_sc as plsc

TB = 2048
ROWS_B = 1024
CHUNK = 32
NBUF = 3
DEPTH = 2
CHUNKS_T = (2048, 4096, 4096, 4096, 2048)
LG = 128


def _argmax_body(x_ref, w1t_ref, b1_ref, w2ta_ref, idx_ref):
    h = jnp.maximum(
        jnp.dot(x_ref[...], w1t_ref[...], preferred_element_type=jnp.float32)
        + b1_ref[...], 0.0)
    ha = jnp.concatenate(
        [h.astype(jnp.bfloat16),
         jnp.ones((h.shape[0], 8), jnp.bfloat16)], axis=1)
    s = jnp.dot(ha, w2ta_ref[...], preferred_element_type=jnp.float32)
    n_grp = s.shape[1] // LG
    val = s[:, 0:LG]
    gidx = jnp.zeros(val.shape, jnp.int32)
    for j in range(1, n_grp):
        v = s[:, j * LG:(j + 1) * LG]
        p = v > val
        val = jnp.where(p, v, val)
        gidx = jnp.where(p, jnp.int32(j), gidx)
    m = jnp.max(val, axis=-1, keepdims=True)
    lane = lax.broadcasted_iota(jnp.int32, val.shape, 1)
    lin = gidx * LG + lane
    idx = jnp.min(jnp.where(val == m, lin, jnp.int32(2**30)), axis=-1)
    idx_ref[0, 0, :] = idx


def _logsoftmax_body(w_ref, b_ref, out_ref):
    z = w_ref[...].T + b_ref[...]
    m = jnp.max(z, axis=-1, keepdims=True)
    e = jnp.exp(z - m)
    lse = m + jnp.log(jnp.sum(e, axis=-1, keepdims=True))
    out_ref[...] = z - lse


def _make_gather(n_tok_chunk, n_tok_total, d, chunk_off, alloc_out):
    info = plsc.get_sparse_core_info()
    nc, ns = info.num_cores, info.num_subcores
    nw = nc * ns
    b_per_w = n_tok_chunk // nw
    n_steps = b_per_w // CHUNK
    mesh = plsc.VectorSubcoreMesh(core_axis_name="c", subcore_axis_name="s")

    @functools.partial(
        pl.kernel,
        mesh=mesh,
        out_type=(jax.ShapeDtypeStruct((n_tok_total, d), jnp.float32)
                  if alloc_out else ()),
        scratch_types=[
            pltpu.VMEM((n_steps, CHUNK), jnp.int32),
            pltpu.VMEM((NBUF, CHUNK, d), jnp.float32),
            pltpu.SemaphoreType.DMA,
            pltpu.SemaphoreType.DMA,
        ],
    )
    def gather_k(table_hbm, idx_hbm, out_hbm, idx_v, rows_v, sem_g, sem_o):
        wid = lax.axis_index("s") * nc + lax.axis_index("c")
        base = chunk_off + wid * b_per_w
        pltpu.sync_copy(idx_hbm.at[wid], idx_v)
        g = [None] * n_steps
        o = [None] * n_steps
        for i in range(min(DEPTH, n_steps)):
            g[i] = pltpu.async_copy(table_hbm.at[idx_v.at[i]],
                                    rows_v.at[i % NBUF], sem_g)
        for i in range(n_steps):
            j = i + DEPTH
            if j < n_steps:
                if j >= NBUF:
                    o[j - NBUF].wait()
                g[j] = pltpu.async_copy(table_hbm.at[idx_v.at[j]],
                                        rows_v.at[j % NBUF], sem_g)
            g[i].wait()
            o[i] = pltpu.async_copy(
                rows_v.at[i % NBUF],
                out_hbm.at[pl.ds(base + i * CHUNK, CHUNK)], sem_o)
        for i in range(max(0, n_steps - NBUF), n_steps):
            o[i].wait()

    return gather_k


def kernel(input, hidden1, W_i2m, b_i2m, W_m2h, b_m2h, W_h2o, b_h2o):
    B, S, I = input.shape
    N = B * S
    H = W_i2m.shape[0]
    C = W_m2h.shape[0]
    O = W_h2o.shape[0]

    x = input.reshape(N, I)
    w1t = W_i2m.T
    b1 = b_i2m.reshape(1, H)
    w2ta = jnp.zeros((H + 8, C), jnp.float32)
    w2ta = w2ta.at[:H].set(W_m2h.T).at[H].set(b_m2h)
    w2ta = w2ta.astype(jnp.bfloat16)

    bo = b_h2o.reshape(1, O)
    table = pl.pallas_call(
        _logsoftmax_body,
        grid=(C // ROWS_B,),
        in_specs=[
            pl.BlockSpec((O, ROWS_B), lambda i: (0, i)),
            pl.BlockSpec((1, O), lambda i: (0, 0)),
        ],
        out_specs=pl.BlockSpec((ROWS_B, O), lambda i: (i, 0)),
        out_shape=jax.ShapeDtypeStruct((C, O), jnp.float32),
    )(W_h2o, bo)

    info = plsc.get_sparse_core_info()
    nw = info.num_cores * info.num_subcores

    out_ref = None
    tok_off = 0
    for c, tpc in enumerate(CHUNKS_T):
        tb = min(TB, tpc)
        nb = tpc // tb
        blk_off = tok_off // tb
        idx3 = pl.pallas_call(
            _argmax_body,
            grid=(nb,),
            in_specs=[
                pl.BlockSpec((tb, I), lambda i, o=blk_off: (i + o, 0)),
                pl.BlockSpec((I, H), lambda i: (0, 0)),
                pl.BlockSpec((1, H), lambda i: (0, 0)),
                pl.BlockSpec((H + 8, C), lambda i: (0, 0)),
            ],
            out_specs=pl.BlockSpec((1, 1, tb), lambda i: (i, 0, 0)),
            out_shape=jax.ShapeDtypeStruct((nb, 1, tb), jnp.int32),
        )(x, w1t, b1, w2ta)
        idx_3d = idx3.reshape(nw, (tpc // nw) // CHUNK, CHUNK)
        if c == 0:
            out0 = _make_gather(tpc, N, O, 0, True)(table, idx_3d)
            out_ref = jax.new_ref(out0)
        else:
            _make_gather(tpc, N, O, tok_off, False)(table, idx_3d, out_ref)
        tok_off += tpc

    return out_ref[...].reshape(B, S, O)


# speedup vs baseline: 1.1239x; 1.1239x over previous
import functools

import jax
import jax.numpy as jnp
from jax import lax
from jax.experimental import pallas as pl
from jax.experimental.pallas import tpu as pltpu
from jax.experimental.pallas import tpu_sc as plsc

TB = 2048
ROWS_B = 1024
CHUNK = 32
NBUF = 3
DEPTH = 2
CHUNKS_T = (4096, 4096, 4096, 4096)
LG = 128


def _argmax_body(x_ref, w1t_ref, b1_ref, w2ta_ref, idx_ref):
    h = jnp.maximum(
        jnp.dot(x_ref[...], w1t_ref[...], preferred_element_type=jnp.float32)
        + b1_ref[...], 0.0)
    ha = jnp.concatenate(
        [h.astype(jnp.bfloat16),
         jnp.ones((h.shape[0], 8), jnp.bfloat16)], axis=1)
    s = jnp.dot(ha, w2ta_ref[...], preferred_element_type=jnp.float32)
    n_grp = s.shape[1] // LG
    val = s[:, 0:LG]
    gidx = jnp.zeros(val.shape, jnp.int32)
    for j in range(1, n_grp):
        v = s[:, j * LG:(j + 1) * LG]
        p = v > val
        val = jnp.where(p, v, val)
        gidx = jnp.where(p, jnp.int32(j), gidx)
    m = jnp.max(val, axis=-1, keepdims=True)
    lane = lax.broadcasted_iota(jnp.int32, val.shape, 1)
    lin = gidx * LG + lane
    idx = jnp.min(jnp.where(val == m, lin, jnp.int32(2**30)), axis=-1)
    idx_ref[0, 0, :] = idx


def _logsoftmax_body(w_ref, b_ref, out_ref):
    z = w_ref[...].T + b_ref[...]
    m = jnp.max(z, axis=-1, keepdims=True)
    e = jnp.exp(z - m)
    lse = m + jnp.log(jnp.sum(e, axis=-1, keepdims=True))
    out_ref[...] = z - lse


def _make_gather(n_tok_chunk, n_tok_total, d, chunk_off, alloc_out):
    info = plsc.get_sparse_core_info()
    nc, ns = info.num_cores, info.num_subcores
    nw = nc * ns
    b_per_w = n_tok_chunk // nw
    n_steps = b_per_w // CHUNK
    mesh = plsc.VectorSubcoreMesh(core_axis_name="c", subcore_axis_name="s")

    @functools.partial(
        pl.kernel,
        mesh=mesh,
        out_type=(jax.ShapeDtypeStruct((n_tok_total, d), jnp.float32)
                  if alloc_out else ()),
        scratch_types=[
            pltpu.VMEM((n_steps, CHUNK), jnp.int32),
            pltpu.VMEM((NBUF, CHUNK, d), jnp.float32),
            pltpu.SemaphoreType.DMA,
            pltpu.SemaphoreType.DMA,
        ],
    )
    def gather_k(table_hbm, idx_hbm, out_hbm, idx_v, rows_v, sem_g, sem_o):
        wid = lax.axis_index("s") * nc + lax.axis_index("c")
        base = chunk_off + wid * b_per_w
        pltpu.sync_copy(idx_hbm.at[wid], idx_v)
        g = [None] * n_steps
        o = [None] * n_steps
        for i in range(min(DEPTH, n_steps)):
            g[i] = pltpu.async_copy(table_hbm.at[idx_v.at[i]],
                                    rows_v.at[i % NBUF], sem_g)
        for i in range(n_steps):
            j = i + DEPTH
            if j < n_steps:
                if j >= NBUF:
                    o[j - NBUF].wait()
                g[j] = pltpu.async_copy(table_hbm.at[idx_v.at[j]],
                                        rows_v.at[j % NBUF], sem_g)
            g[i].wait()
            o[i] = pltpu.async_copy(
                rows_v.at[i % NBUF],
                out_hbm.at[pl.ds(base + i * CHUNK, CHUNK)], sem_o)
        for i in range(max(0, n_steps - NBUF), n_steps):
            o[i].wait()

    return gather_k


def kernel(input, hidden1, W_i2m, b_i2m, W_m2h, b_m2h, W_h2o, b_h2o):
    B, S, I = input.shape
    N = B * S
    H = W_i2m.shape[0]
    C = W_m2h.shape[0]
    O = W_h2o.shape[0]

    x = input.reshape(N, I)
    w1t = W_i2m.T
    b1 = b_i2m.reshape(1, H)
    w2ta = jnp.zeros((H + 8, C), jnp.float32)
    w2ta = w2ta.at[:H].set(W_m2h.T).at[H].set(b_m2h)
    w2ta = w2ta.astype(jnp.bfloat16)

    bo = b_h2o.reshape(1, O)
    table = pl.pallas_call(
        _logsoftmax_body,
        grid=(C // ROWS_B,),
        in_specs=[
            pl.BlockSpec((O, ROWS_B), lambda i: (0, i)),
            pl.BlockSpec((1, O), lambda i: (0, 0)),
        ],
        out_specs=pl.BlockSpec((ROWS_B, O), lambda i: (i, 0)),
        out_shape=jax.ShapeDtypeStruct((C, O), jnp.float32),
    )(W_h2o, bo)

    info = plsc.get_sparse_core_info()
    nw = info.num_cores * info.num_subcores

    out_ref = None
    tok_off = 0
    for c, tpc in enumerate(CHUNKS_T):
        tb = min(TB, tpc)
        nb = tpc // tb
        blk_off = tok_off // tb
        idx3 = pl.pallas_call(
            _argmax_body,
            grid=(nb,),
            in_specs=[
                pl.BlockSpec((tb, I), lambda i, o=blk_off: (i + o, 0)),
                pl.BlockSpec((I, H), lambda i: (0, 0)),
                pl.BlockSpec((1, H), lambda i: (0, 0)),
                pl.BlockSpec((H + 8, C), lambda i: (0, 0)),
            ],
            out_specs=pl.BlockSpec((1, 1, tb), lambda i: (i, 0, 0)),
            out_shape=jax.ShapeDtypeStruct((nb, 1, tb), jnp.int32),
        )(x, w1t, b1, w2ta)
        idx_3d = idx3.reshape(nw, (tpc // nw) // CHUNK, CHUNK)
        if c == 0:
            out0 = _make_gather(tpc, N, O, 0, True)(table, idx_3d)
            out_ref = jax.new_ref(out0)
        else:
            _make_gather(tpc, N, O, tok_off, False)(table, idx_3d, out_ref)
        tok_off += tpc

    return out_ref[...].reshape(B, S, O)
